# baseline (speedup 1.0000x reference)
import functools
import numpy as np
import jax
import jax.numpy as jnp
from jax import lax
from jax.experimental import pallas as pl
from jax.experimental.pallas import tpu as pltpu
from jax.experimental.pallas import tpu_sc as plsc

B = 4
LQ = 1024
D_MODEL = 256
D_HEAD = 64
N_LEVELS = 3
N_HEADS = 6
N_POINTS = 9
SHAPES = [(64, 64), (32, 32), (16, 16)]
LIN = sum(h * w for h, w in SHAPES)
NQ = B * LQ
LP = 32
KROWS = 4 * LP
ITEMS = N_HEADS * NQ
NW = 32
IPW = ITEMS // NW
CHUNK = 128
NCHUNK = IPW // CHUNK

_lvl_of_lp = np.array([lp // N_POINTS if lp < 27 else 0 for lp in range(LP)])
_W_lp = np.array([SHAPES[l][1] for l in _lvl_of_lp], np.float32)
_H_lp = np.array([SHAPES[l][0] for l in _lvl_of_lp], np.float32)
_starts = np.cumsum([0] + [h * w for h, w in SHAPES])[:-1]
_start_lp = np.array([_starts[l] for l in _lvl_of_lp], np.int32)


def _val_body(src_ref, w_ref, b_ref, out_ref):
    out_ref[0] = jnp.dot(src_ref[...], w_ref[0],
                         preferred_element_type=jnp.float32) + b_ref[0]


def _value_table(src_flat, W_val, b_val):
    RB = 512
    grid = (N_HEADS, (B * LIN) // RB)
    return pl.pallas_call(
        _val_body,
        grid=grid,
        in_specs=[
            pl.BlockSpec((RB, D_MODEL), lambda h, r: (r, 0)),
            pl.BlockSpec((1, D_MODEL, D_HEAD), lambda h, r: (h, 0, 0)),
            pl.BlockSpec((1, 1, D_HEAD), lambda h, r: (h, 0, 0)),
        ],
        out_specs=pl.BlockSpec((1, RB, D_HEAD), lambda h, r: (h, r, 0)),
        out_shape=jax.ShapeDtypeStruct((N_HEADS, B * LIN, D_HEAD), jnp.float32),
    )(src_flat,
      W_val.reshape(D_MODEL, N_HEADS, D_HEAD).transpose(1, 0, 2),
      b_val.reshape(N_HEADS, 1, D_HEAD))


def _samp_body(tgt_ref, qpos_ref, wx_ref, wy_ref, wa_ref,
               bx_ref, by_ref, ba_ref, rpx_ref, rpy_ref,
               cf_ref, ci_ref, idx_ref, w_ref):
    h = pl.program_id(1)
    b = pl.program_id(0) // (LQ // 256)
    q = tgt_ref[...] + qpos_ref[...]
    offx = jnp.dot(q, wx_ref[0], preferred_element_type=jnp.float32) + bx_ref[0]
    offy = jnp.dot(q, wy_ref[0], preferred_element_type=jnp.float32) + by_ref[0]
    a = jnp.dot(q, wa_ref[0], preferred_element_type=jnp.float32) + ba_ref[0]
    m = jnp.max(a, axis=-1, keepdims=True)
    e = jnp.exp(a - m)
    aw = e / jnp.sum(e, axis=-1, keepdims=True)

    cW = cf_ref[0]
    cH = cf_ref[1]
    cWi = ci_ref[0]
    cS = ci_ref[1]

    x = (rpx_ref[...] + offx) * cW - 0.5
    y = (rpy_ref[...] + offy) * cH - 0.5
    x0 = jnp.floor(x)
    y0 = jnp.floor(y)
    wx1 = x - x0
    wx0 = 1.0 - wx1
    wy1 = y - y0
    wy0 = 1.0 - wy1
    base = (h * B + b) * LIN

    def corner(yy, xx, wgt):
        valid = (xx >= 0) & (xx <= cW - 1) & (yy >= 0) & (yy <= cH - 1)
        xc = jnp.clip(xx, 0, cW - 1).astype(jnp.int32)
        yc = jnp.clip(yy, 0, cH - 1).astype(jnp.int32)
        return base + cS + yc * cWi + xc, aw * wgt * valid.astype(jnp.float32)

    i0, w0 = corner(y0, x0, wy0 * wx0)
    i1, w1 = corner(y0, x0 + 1.0, wy0 * wx1)
    i2, w2 = corner(y0 + 1.0, x0, wy1 * wx0)
    i3, w3 = corner(y0 + 1.0, x0 + 1.0, wy1 * wx1)
    idx_ref[0] = jnp.concatenate([i0, i1, i2, i3], axis=-1)
    w_ref[0] = jnp.concatenate([w0, w1, w2, w3], axis=-1)


def _sampling(tgt_flat, qpos_flat, Wx, Wy, Wa, bx, by, ba, rpx, rpy):
    QB = 256
    grid = (NQ // QB, N_HEADS)
    cf = jnp.stack([jnp.asarray(_W_lp), jnp.asarray(_H_lp)]).reshape(2, 1, LP)
    ci = jnp.stack([jnp.asarray(_W_lp.astype(np.int32)),
                    jnp.asarray(_start_lp)]).reshape(2, 1, LP)
    return pl.pallas_call(
        _samp_body,
        grid=grid,
        in_specs=[
            pl.BlockSpec((QB, D_MODEL), lambda r, h: (r, 0)),
            pl.BlockSpec((QB, D_MODEL), lambda r, h: (r, 0)),
            pl.BlockSpec((1, D_MODEL, LP), lambda r, h: (h, 0, 0)),
            pl.BlockSpec((1, D_MODEL, LP), lambda r, h: (h, 0, 0)),
            pl.BlockSpec((1, D_MODEL, LP), lambda r, h: (h, 0, 0)),
            pl.BlockSpec((1, 1, LP), lambda r, h: (h, 0, 0)),
            pl.BlockSpec((1, 1, LP), lambda r, h: (h, 0, 0)),
            pl.BlockSpec((1, 1, LP), lambda r, h: (h, 0, 0)),
            pl.BlockSpec((QB, LP), lambda r, h: (r, 0)),
            pl.BlockSpec((QB, LP), lambda r, h: (r, 0)),
            pl.BlockSpec((2, 1, LP), lambda r, h: (0, 0, 0)),
            pl.BlockSpec((2, 1, LP), lambda r, h: (0, 0, 0)),
        ],
        out_specs=[
            pl.BlockSpec((1, QB, KROWS), lambda r, h: (h, r, 0)),
            pl.BlockSpec((1, QB, KROWS), lambda r, h: (h, r, 0)),
        ],
        out_shape=[
            jax.ShapeDtypeStruct((N_HEADS, NQ, KROWS), jnp.int32),
            jax.ShapeDtypeStruct((N_HEADS, NQ, KROWS), jnp.float32),
        ],
    )(tgt_flat, qpos_flat, Wx, Wy, Wa, bx, by, ba, rpx, rpy, cf, ci)


def _sc_gather(table, idx_all, w_flat):
    mesh = plsc.VectorSubcoreMesh(core_axis_name="c", subcore_axis_name="s")

    @functools.partial(
        pl.kernel,
        mesh=mesh,
        compiler_params=pltpu.CompilerParams(use_tc_tiling_on_sc=False),
        out_type=jax.ShapeDtypeStruct((ITEMS, D_HEAD), jnp.float32),
        scratch_types=[
            pltpu.VMEM((CHUNK * KROWS,), jnp.int32),
            pltpu.VMEM((CHUNK * KROWS,), jnp.float32),
            pltpu.VMEM((KROWS // 2, D_HEAD), jnp.float32),
            pltpu.VMEM((KROWS // 2, D_HEAD), jnp.float32),
            pltpu.VMEM((CHUNK, D_HEAD), jnp.float32),
            pltpu.SemaphoreType.DMA,
            pltpu.SemaphoreType.DMA,
        ],
    )
    def sc_k(table_hbm, idx_hbm, w_hbm, out_hbm, idx_v, w_v,
             rows_a, rows_b, obuf_v, sem_a, sem_b):
        wid = lax.axis_index("s") * 2 + lax.axis_index("c")
        lane = jnp.arange(16, dtype=jnp.int32)
        bcast_idx = [lane * 0 + t for t in range(16)]

        def lane_bcast(wv, t):
            return lax.gather(
                wv, bcast_idx[t][:, None],
                lax.GatherDimensionNumbers(
                    offset_dims=(), collapsed_slice_dims=(0,),
                    start_index_map=(0,)),
                slice_sizes=(1,),
                mode=lax.GatherScatterMode.PROMISE_IN_BOUNDS)

        HR = KROWS // 2

        def gather_half(elem_off, rows, sem):
            pltpu.async_copy(
                table_hbm.at[idx_v.at[pl.ds(elem_off, HR)]], rows, sem)

        def accum_half(w_off, rows, acc):
            for c in range(HR // 16):
                wv = w_v[pl.ds(w_off + c * 16, 16)]
                for t in range(16):
                    wj = lane_bcast(wv, t)
                    r = c * 16 + t
                    for k in range(4):
                        acc[k] = acc[k] + wj * rows[r, pl.ds(k * 16, 16)]
            return acc

        def wait_on(rows, sem):
            pltpu.make_async_copy(table_hbm.at[pl.ds(0, HR)], rows,
                                  sem).wait()

        for chunk in range(NCHUNK):
            cbase = wid * IPW + chunk * CHUNK
            pltpu.sync_copy(idx_hbm.at[pl.ds(cbase * KROWS, CHUNK * KROWS)],
                            idx_v)
            pltpu.sync_copy(w_hbm.at[pl.ds(cbase * KROWS, CHUNK * KROWS)], w_v)
            gather_half(0, rows_a, sem_a)

            def item_body(j, _):
                base = j * KROWS
                wait_on(rows_a, sem_a)
                gather_half(base + HR, rows_b, sem_b)
                acc = [jnp.zeros((16,), jnp.float32) for _ in range(4)]
                acc = accum_half(base, rows_a, acc)
                wait_on(rows_b, sem_b)
                gather_half(jnp.minimum(base + KROWS, (CHUNK - 1) * KROWS),
                            rows_a, sem_a)
                acc = accum_half(base + HR, rows_b, acc)
                for k in range(4):
                    obuf_v[j, pl.ds(k * 16, 16)] = acc[k]
                return 0

            lax.fori_loop(0, CHUNK, item_body, 0)
            wait_on(rows_a, sem_a)
            pltpu.sync_copy(obuf_v, out_hbm.at[pl.ds(cbase, CHUNK)])

    return sc_k(table, idx_all, w_flat)


def _out_body(v_ref, w_ref, b_ref, out_ref):
    out_ref[...] = jnp.dot(v_ref[...], w_ref[...],
                           preferred_element_type=jnp.float32) + b_ref[0]


def _out_proj(vstack, W_out, b_out):
    RB = 512
    return pl.pallas_call(
        _out_body,
        grid=(NQ // RB,),
        in_specs=[
            pl.BlockSpec((RB, N_HEADS * D_HEAD), lambda r: (r, 0)),
            pl.BlockSpec((N_HEADS * D_HEAD, D_MODEL), lambda r: (0, 0)),
            pl.BlockSpec((1, 1, D_MODEL), lambda r: (0, 0, 0)),
        ],
        out_specs=pl.BlockSpec((RB, D_MODEL), lambda r: (r, 0)),
        out_shape=jax.ShapeDtypeStruct((NQ, D_MODEL), jnp.float32),
    )(vstack, W_out, b_out.reshape(1, 1, D_MODEL))


def kernel(tgt, src, query_pos, reference_points, src_spatial_shapes,
           level_start_index, src_padding_mask, W_off, b_off, W_attn, b_attn,
           W_val, b_val, W_out, b_out):
    src_flat = src.reshape(B * LIN, D_MODEL)
    table = _value_table(src_flat, W_val, b_val).reshape(
        N_HEADS * B * LIN, D_HEAD)

    Wo = W_off.reshape(D_MODEL, N_HEADS, N_LEVELS, N_POINTS, 2)
    bo = b_off.reshape(N_HEADS, N_LEVELS, N_POINTS, 2)
    norm = np.array([[w, h] for h, w in SHAPES], np.float32)
    Wo = Wo / norm[None, None, :, None, :]
    bo = bo / norm[None, :, None, :]
    Wx = jnp.zeros((D_MODEL, N_HEADS, LP), jnp.float32)
    Wy = jnp.zeros((D_MODEL, N_HEADS, LP), jnp.float32)
    bx = jnp.zeros((N_HEADS, LP), jnp.float32)
    by = jnp.zeros((N_HEADS, LP), jnp.float32)
    Wx = Wx.at[:, :, :27].set(Wo[..., 0].reshape(D_MODEL, N_HEADS, 27))
    Wy = Wy.at[:, :, :27].set(Wo[..., 1].reshape(D_MODEL, N_HEADS, 27))
    bx = bx.at[:, :27].set(bo[..., 0].reshape(N_HEADS, 27))
    by = by.at[:, :27].set(bo[..., 1].reshape(N_HEADS, 27))
    Wa = jnp.zeros((D_MODEL, N_HEADS, LP), jnp.float32)
    ba = jnp.full((N_HEADS, LP), -1e30, jnp.float32)
    Wa = Wa.at[:, :, :27].set(W_attn.reshape(D_MODEL, N_HEADS, 27))
    ba = ba.at[:, :27].set(b_attn.reshape(N_HEADS, 27))
    Wx = Wx.transpose(1, 0, 2)
    Wy = Wy.transpose(1, 0, 2)
    Wa = Wa.transpose(1, 0, 2)

    rp = reference_points.reshape(NQ, N_LEVELS, 2)
    lvl_map = jnp.asarray(_lvl_of_lp.astype(np.int32))
    rpx = rp[:, :, 0][:, lvl_map]
    rpy = rp[:, :, 1][:, lvl_map]

    idx_all, w_all = _sampling(
        tgt.reshape(NQ, D_MODEL), query_pos.reshape(NQ, D_MODEL),
        Wx, Wy, Wa,
        bx.reshape(N_HEADS, 1, LP), by.reshape(N_HEADS, 1, LP),
        ba.reshape(N_HEADS, 1, LP), rpx, rpy)

    out_sc = _sc_gather(table, idx_all.reshape(ITEMS * KROWS),
                        w_all.reshape(ITEMS * KROWS))

    vstack = out_sc.reshape(N_HEADS, NQ, D_HEAD).transpose(1, 0, 2).reshape(
        NQ, N_HEADS * D_HEAD)
    out = _out_proj(vstack, W_out, b_out)
    return out.reshape(B, LQ, D_MODEL)


# speedup vs baseline: 1.8002x; 1.8002x over previous
import functools
import numpy as np
import jax
import jax.numpy as jnp
from jax import lax
from jax.experimental import pallas as pl
from jax.experimental.pallas import tpu as pltpu
from jax.experimental.pallas import tpu_sc as plsc

B = 4
LQ = 1024
D_MODEL = 256
D_HEAD = 64
N_LEVELS = 3
N_HEADS = 6
N_POINTS = 9
SHAPES = [(64, 64), (32, 32), (16, 16)]
LIN = sum(h * w for h, w in SHAPES)
NQ = B * LQ
LP = 28
KROWS = 4 * LP
ITEMS = N_HEADS * NQ
NW = 32
IPW = ITEMS // NW
CHUNK = 128
NCHUNK = IPW // CHUNK

_lvl_of_lp = np.array([lp // N_POINTS if lp < 27 else 0 for lp in range(LP)])
_W_lp = np.array([SHAPES[l][1] for l in _lvl_of_lp], np.float32)
_H_lp = np.array([SHAPES[l][0] for l in _lvl_of_lp], np.float32)
_starts = np.cumsum([0] + [h * w for h, w in SHAPES])[:-1]
_start_lp = np.array([_starts[l] for l in _lvl_of_lp], np.int32)


def _val_body(src_ref, w_ref, b_ref, out_ref):
    out_ref[0] = jnp.dot(src_ref[...], w_ref[0],
                         preferred_element_type=jnp.float32) + b_ref[0]


def _value_table(src_flat, W_val, b_val):
    RB = 512
    grid = (N_HEADS, (B * LIN) // RB)
    return pl.pallas_call(
        _val_body,
        grid=grid,
        in_specs=[
            pl.BlockSpec((RB, D_MODEL), lambda h, r: (r, 0)),
            pl.BlockSpec((1, D_MODEL, D_HEAD), lambda h, r: (h, 0, 0)),
            pl.BlockSpec((1, 1, D_HEAD), lambda h, r: (h, 0, 0)),
        ],
        out_specs=pl.BlockSpec((1, RB, D_HEAD), lambda h, r: (h, r, 0)),
        out_shape=jax.ShapeDtypeStruct((N_HEADS, B * LIN, D_HEAD), jnp.float32),
    )(src_flat,
      W_val.reshape(D_MODEL, N_HEADS, D_HEAD).transpose(1, 0, 2),
      b_val.reshape(N_HEADS, 1, D_HEAD))


def _samp_body(tgt_ref, qpos_ref, wx_ref, wy_ref, wa_ref,
               bx_ref, by_ref, ba_ref, rpx_ref, rpy_ref,
               cf_ref, ci_ref, idx_ref, w_ref):
    h = pl.program_id(1)
    b = pl.program_id(0) // (LQ // 256)
    q = tgt_ref[...] + qpos_ref[...]
    offx = jnp.dot(q, wx_ref[0], preferred_element_type=jnp.float32) + bx_ref[0]
    offy = jnp.dot(q, wy_ref[0], preferred_element_type=jnp.float32) + by_ref[0]
    a = jnp.dot(q, wa_ref[0], preferred_element_type=jnp.float32) + ba_ref[0]
    m = jnp.max(a, axis=-1, keepdims=True)
    e = jnp.exp(a - m)
    aw = e / jnp.sum(e, axis=-1, keepdims=True)

    cW = cf_ref[0]
    cH = cf_ref[1]
    cWi = ci_ref[0]
    cS = ci_ref[1]

    x = (rpx_ref[...] + offx) * cW - 0.5
    y = (rpy_ref[...] + offy) * cH - 0.5
    x0 = jnp.floor(x)
    y0 = jnp.floor(y)
    wx1 = x - x0
    wx0 = 1.0 - wx1
    wy1 = y - y0
    wy0 = 1.0 - wy1
    base = (h * B + b) * LIN

    def corner(yy, xx, wgt):
        valid = (xx >= 0) & (xx <= cW - 1) & (yy >= 0) & (yy <= cH - 1)
        xc = jnp.clip(xx, 0, cW - 1).astype(jnp.int32)
        yc = jnp.clip(yy, 0, cH - 1).astype(jnp.int32)
        return base + cS + yc * cWi + xc, aw * wgt * valid.astype(jnp.float32)

    i0, w0 = corner(y0, x0, wy0 * wx0)
    i1, w1 = corner(y0, x0 + 1.0, wy0 * wx1)
    i2, w2 = corner(y0 + 1.0, x0, wy1 * wx0)
    i3, w3 = corner(y0 + 1.0, x0 + 1.0, wy1 * wx1)
    idx_ref[0] = jnp.concatenate([i0, i1, i2, i3], axis=-1)
    w_ref[0] = jnp.concatenate([w0, w1, w2, w3], axis=-1)


def _sampling(tgt_flat, qpos_flat, Wx, Wy, Wa, bx, by, ba, rpx, rpy):
    QB = 256
    grid = (NQ // QB, N_HEADS)
    cf = jnp.stack([jnp.asarray(_W_lp), jnp.asarray(_H_lp)]).reshape(2, 1, LP)
    ci = jnp.stack([jnp.asarray(_W_lp.astype(np.int32)),
                    jnp.asarray(_start_lp)]).reshape(2, 1, LP)
    return pl.pallas_call(
        _samp_body,
        grid=grid,
        in_specs=[
            pl.BlockSpec((QB, D_MODEL), lambda r, h: (r, 0)),
            pl.BlockSpec((QB, D_MODEL), lambda r, h: (r, 0)),
            pl.BlockSpec((1, D_MODEL, LP), lambda r, h: (h, 0, 0)),
            pl.BlockSpec((1, D_MODEL, LP), lambda r, h: (h, 0, 0)),
            pl.BlockSpec((1, D_MODEL, LP), lambda r, h: (h, 0, 0)),
            pl.BlockSpec((1, 1, LP), lambda r, h: (h, 0, 0)),
            pl.BlockSpec((1, 1, LP), lambda r, h: (h, 0, 0)),
            pl.BlockSpec((1, 1, LP), lambda r, h: (h, 0, 0)),
            pl.BlockSpec((QB, LP), lambda r, h: (r, 0)),
            pl.BlockSpec((QB, LP), lambda r, h: (r, 0)),
            pl.BlockSpec((2, 1, LP), lambda r, h: (0, 0, 0)),
            pl.BlockSpec((2, 1, LP), lambda r, h: (0, 0, 0)),
        ],
        out_specs=[
            pl.BlockSpec((1, QB, KROWS), lambda r, h: (h, r, 0)),
            pl.BlockSpec((1, QB, KROWS), lambda r, h: (h, r, 0)),
        ],
        out_shape=[
            jax.ShapeDtypeStruct((N_HEADS, NQ, KROWS), jnp.int32),
            jax.ShapeDtypeStruct((N_HEADS, NQ, KROWS), jnp.float32),
        ],
    )(tgt_flat, qpos_flat, Wx, Wy, Wa, bx, by, ba, rpx, rpy, cf, ci)


def _sc_gather(table, idx_all, w_flat):
    mesh = plsc.VectorSubcoreMesh(core_axis_name="c", subcore_axis_name="s")

    @functools.partial(
        pl.kernel,
        mesh=mesh,
        compiler_params=pltpu.CompilerParams(use_tc_tiling_on_sc=False),
        out_type=jax.ShapeDtypeStruct((ITEMS, D_HEAD), jnp.float32),
        scratch_types=[
            pltpu.VMEM((CHUNK * KROWS,), jnp.int32),
            pltpu.VMEM((CHUNK * KROWS,), jnp.float32),
            pltpu.VMEM((2, KROWS, D_HEAD), jnp.float32),
            pltpu.VMEM((CHUNK, D_HEAD), jnp.float32),
            pltpu.SemaphoreType.DMA,
            pltpu.SemaphoreType.DMA,
        ],
    )
    def sc_k(table_hbm, idx_hbm, w_hbm, out_hbm, idx_v, w_v,
             rows_v, obuf_v, sem_a, sem_b):
        wid = lax.axis_index("s") * 2 + lax.axis_index("c")
        lane = jnp.arange(16, dtype=jnp.int32)
        bcast_idx = [lane * 0 + t for t in range(16)]

        def lane_bcast(wv, t):
            return lax.gather(
                wv, bcast_idx[t][:, None],
                lax.GatherDimensionNumbers(
                    offset_dims=(), collapsed_slice_dims=(0,),
                    start_index_map=(0,)),
                slice_sizes=(1,),
                mode=lax.GatherScatterMode.PROMISE_IN_BOUNDS)

        def gather_item(elem_off, slot, sem):
            pltpu.async_copy(
                table_hbm.at[idx_v.at[pl.ds(elem_off, KROWS)]],
                rows_v.at[slot], sem)

        def wait_sem(sem):
            pltpu.make_async_copy(table_hbm.at[pl.ds(0, KROWS)],
                                  rows_v.at[0], sem).wait()

        for chunk in range(NCHUNK):
            cbase = wid * IPW + chunk * CHUNK
            pltpu.sync_copy(idx_hbm.at[pl.ds(cbase * KROWS, CHUNK * KROWS)],
                            idx_v)
            pltpu.sync_copy(w_hbm.at[pl.ds(cbase * KROWS, CHUNK * KROWS)], w_v)
            gather_item(0, 0, sem_a)

            def item_body(j, _):
                par = j % 2
                nxt_off = jnp.minimum((j + 1) * KROWS, (CHUNK - 1) * KROWS)

                @pl.when((j + 1) % 2 == 0)
                def _():
                    gather_item(nxt_off, 0, sem_a)

                @pl.when((j + 1) % 2 == 1)
                def _():
                    gather_item(nxt_off, 1, sem_b)

                @pl.when(par == 0)
                def _():
                    wait_sem(sem_a)

                @pl.when(par == 1)
                def _():
                    wait_sem(sem_b)

                acc = [jnp.zeros((16,), jnp.float32) for _ in range(4)]
                for c in range(KROWS // 16):
                    wv = w_v[pl.ds(j * KROWS + c * 16, 16)]
                    for t in range(16):
                        wj = lane_bcast(wv, t)
                        r = c * 16 + t
                        for k in range(4):
                            acc[k] = acc[k] + wj * rows_v[par, r,
                                                          pl.ds(k * 16, 16)]
                for k in range(4):
                    obuf_v[j, pl.ds(k * 16, 16)] = acc[k]
                return 0

            lax.fori_loop(0, CHUNK, item_body, 0)
            wait_sem(sem_a)
            pltpu.sync_copy(obuf_v, out_hbm.at[pl.ds(cbase, CHUNK)])

    return sc_k(table, idx_all, w_flat)


def _out_body(v_ref, w_ref, b_ref, out_ref):
    out_ref[...] = jnp.dot(v_ref[...], w_ref[...],
                           preferred_element_type=jnp.float32) + b_ref[0]


def _out_proj(vstack, W_out, b_out):
    RB = 512
    return pl.pallas_call(
        _out_body,
        grid=(NQ // RB,),
        in_specs=[
            pl.BlockSpec((RB, N_HEADS * D_HEAD), lambda r: (r, 0)),
            pl.BlockSpec((N_HEADS * D_HEAD, D_MODEL), lambda r: (0, 0)),
            pl.BlockSpec((1, 1, D_MODEL), lambda r: (0, 0, 0)),
        ],
        out_specs=pl.BlockSpec((RB, D_MODEL), lambda r: (r, 0)),
        out_shape=jax.ShapeDtypeStruct((NQ, D_MODEL), jnp.float32),
    )(vstack, W_out, b_out.reshape(1, 1, D_MODEL))


def kernel(tgt, src, query_pos, reference_points, src_spatial_shapes,
           level_start_index, src_padding_mask, W_off, b_off, W_attn, b_attn,
           W_val, b_val, W_out, b_out):
    src_flat = src.reshape(B * LIN, D_MODEL)
    table = _value_table(src_flat, W_val, b_val).reshape(
        N_HEADS * B * LIN, D_HEAD)

    Wo = W_off.reshape(D_MODEL, N_HEADS, N_LEVELS, N_POINTS, 2)
    bo = b_off.reshape(N_HEADS, N_LEVELS, N_POINTS, 2)
    norm = np.array([[w, h] for h, w in SHAPES], np.float32)
    Wo = Wo / norm[None, None, :, None, :]
    bo = bo / norm[None, :, None, :]
    Wx = jnp.zeros((D_MODEL, N_HEADS, LP), jnp.float32)
    Wy = jnp.zeros((D_MODEL, N_HEADS, LP), jnp.float32)
    bx = jnp.zeros((N_HEADS, LP), jnp.float32)
    by = jnp.zeros((N_HEADS, LP), jnp.float32)
    Wx = Wx.at[:, :, :27].set(Wo[..., 0].reshape(D_MODEL, N_HEADS, 27))
    Wy = Wy.at[:, :, :27].set(Wo[..., 1].reshape(D_MODEL, N_HEADS, 27))
    bx = bx.at[:, :27].set(bo[..., 0].reshape(N_HEADS, 27))
    by = by.at[:, :27].set(bo[..., 1].reshape(N_HEADS, 27))
    Wa = jnp.zeros((D_MODEL, N_HEADS, LP), jnp.float32)
    ba = jnp.full((N_HEADS, LP), -1e30, jnp.float32)
    Wa = Wa.at[:, :, :27].set(W_attn.reshape(D_MODEL, N_HEADS, 27))
    ba = ba.at[:, :27].set(b_attn.reshape(N_HEADS, 27))
    Wx = Wx.transpose(1, 0, 2)
    Wy = Wy.transpose(1, 0, 2)
    Wa = Wa.transpose(1, 0, 2)

    rp = reference_points.reshape(NQ, N_LEVELS, 2)
    lvl_map = jnp.asarray(_lvl_of_lp.astype(np.int32))
    rpx = rp[:, :, 0][:, lvl_map]
    rpy = rp[:, :, 1][:, lvl_map]

    idx_all, w_all = _sampling(
        tgt.reshape(NQ, D_MODEL), query_pos.reshape(NQ, D_MODEL),
        Wx, Wy, Wa,
        bx.reshape(N_HEADS, 1, LP), by.reshape(N_HEADS, 1, LP),
        ba.reshape(N_HEADS, 1, LP), rpx, rpy)

    out_sc = _sc_gather(table, idx_all.reshape(ITEMS * KROWS),
                        w_all.reshape(ITEMS * KROWS))

    vstack = out_sc.reshape(N_HEADS, NQ, D_HEAD).transpose(1, 0, 2).reshape(
        NQ, N_HEADS * D_HEAD)
    out = _out_proj(vstack, W_out, b_out)
    return out.reshape(B, LQ, D_MODEL)
